# Initial kernel scaffold; baseline (speedup 1.0000x reference)
import jax
import jax.numpy as jnp
from jax.experimental import pallas as pl


def kernel(o, x, oW1, ob1, oW2, ob2, eW1, eb1, eW2, eb2):
    raise NotImplementedError("write your pallas kernel here")


# speedup vs baseline: 5.3041x; 5.3041x over previous
import jax
import jax.numpy as jnp
from jax.experimental import pallas as pl
from jax.experimental.pallas import tpu as pltpu

_B, _T, _DIN, _H, _C = 1024, 85, 20, 1024, 27
_OBJ = _T * 36


def _base_kernel(o_ref, oW1_ref, ob1_ref, oW2_ref, ob2_ref, eW1h_ref,
                 eb1_ref, base_ref):
    h = jnp.dot(o_ref[...], oW1_ref[...],
                preferred_element_type=jnp.float32) + ob1_ref[...]
    h = jnp.maximum(h, 0.0)
    obj = jnp.dot(h, oW2_ref[...],
                  preferred_element_type=jnp.float32) + ob2_ref[...]
    base_ref[...] = jnp.dot(obj, eW1h_ref[...],
                            preferred_element_type=jnp.float32) + eb1_ref[...]


def _loop_kernel(xT_ref, base_ref, w1_ref, w2_ref, eb2_ref, out_ref,
                 prev_ref):
    t = pl.program_id(1)
    xt = xT_ref[0]

    @pl.when(t == 0)
    def _():
        prev_ref[...] = xt[:, 0:1]

    cur = jnp.concatenate([prev_ref[...], xt[:, 1:]], axis=1)
    h = jnp.dot(cur, w1_ref[...],
                preferred_element_type=jnp.float32) + base_ref[...]
    h = jnp.maximum(h, 0.0)
    ew = jnp.dot(h, w2_ref[...],
                 preferred_element_type=jnp.float32) + eb2_ref[...]
    prev_ref[...] = ew[:, 0:1]
    out_ref[0] = ew


def kernel(o, x, oW1, ob1, oW2, ob2, eW1, eb1, eW2, eb2):
    MB = 256
    base = pl.pallas_call(
        _base_kernel,
        grid=(_B // MB,),
        in_specs=[
            pl.BlockSpec((MB, _OBJ), lambda i: (i, 0)),
            pl.BlockSpec((_OBJ, _H), lambda i: (0, 0)),
            pl.BlockSpec((1, _H), lambda i: (0, 0)),
            pl.BlockSpec((_H, _H), lambda i: (0, 0)),
            pl.BlockSpec((1, _H), lambda i: (0, 0)),
            pl.BlockSpec((_H, _H), lambda i: (0, 0)),
            pl.BlockSpec((1, _H), lambda i: (0, 0)),
        ],
        out_specs=pl.BlockSpec((MB, _H), lambda i: (i, 0)),
        out_shape=jax.ShapeDtypeStruct((_B, _H), jnp.float32),
        compiler_params=pltpu.CompilerParams(
            dimension_semantics=("parallel",),
            vmem_limit_bytes=56 * 1024 * 1024,
        ),
        name="walk_base",
    )(o, oW1, ob1.reshape(1, -1), oW2, ob2.reshape(1, -1), eW1[_DIN:],
      eb1.reshape(1, -1))

    xT = jnp.swapaxes(x, 0, 1)
    BB = 512
    outT = pl.pallas_call(
        _loop_kernel,
        grid=(_B // BB, _T - 1),
        in_specs=[
            pl.BlockSpec((1, BB, _DIN), lambda b, t: (t, b, 0)),
            pl.BlockSpec((BB, _H), lambda b, t: (b, 0)),
            pl.BlockSpec((_DIN, _H), lambda b, t: (0, 0)),
            pl.BlockSpec((_H, _C), lambda b, t: (0, 0)),
            pl.BlockSpec((1, _C), lambda b, t: (0, 0)),
        ],
        out_specs=pl.BlockSpec((1, BB, _C), lambda b, t: (t, b, 0)),
        out_shape=jax.ShapeDtypeStruct((_T - 1, _B, _C), jnp.float32),
        scratch_shapes=[pltpu.VMEM((BB, 1), jnp.float32)],
        compiler_params=pltpu.CompilerParams(
            dimension_semantics=("parallel", "arbitrary"),
        ),
        name="walk_loop",
    )(xT, base, eW1[:_DIN], eW2, eb2.reshape(1, -1))
    return jnp.swapaxes(outT, 0, 1)
